# baseline (speedup 1.0000x reference)
import jax
import jax.numpy as jnp
from jax.experimental import pallas as pl
from jax.experimental.pallas import tpu as pltpu

B = 262144
R = B // 4
RB1 = 4096
NB1 = R // RB1
D = B * 4 // 128
RB2 = 4096
NB2 = D // RB2


def _mlp_body(x_ref, w1_ref, b1_ref, w2_ref, b2_ref, w3_ref, b3_ref,
              logits_ref, sums_ref):
    xb = x_ref[...].astype(jnp.bfloat16)
    h = jnp.dot(xb, w1_ref[...], preferred_element_type=jnp.float32)
    h = jnp.maximum(h + b1_ref[...], 0.0).astype(jnp.bfloat16)
    h = jnp.tanh(jnp.dot(h, w2_ref[...], preferred_element_type=jnp.float32)
                 + b2_ref[...]).astype(jnp.bfloat16)
    z = jnp.dot(h, w3_ref[...], preferred_element_type=jnp.float32) + b3_ref[...]
    zt = jax.lax.transpose(z, (1, 0))
    logits_ref[...] = zt
    sums_ref[...] = jnp.full((1, 1, 8), jnp.sum(jnp.exp(zt)), jnp.float32)


def _norm_body(z_ref, sums_ref, out_ref):
    s = jnp.sum(sums_ref[...]) * 0.125
    out_ref[...] = jnp.exp(z_ref[...]) / s


def _blockdiag4(w):
    k, n = w.shape
    out = jnp.zeros((4 * k, 4 * n), w.dtype)
    for g in range(4):
        out = out.at[g * k:(g + 1) * k, g * n:(g + 1) * n].set(w)
    return out


@jax.jit
def kernel(x, wz1, b1, wz2, b2, wz3, b3):
    full = lambda *_: (0, 0)
    full3 = lambda *_: (0, 0, 0)
    bf = jnp.bfloat16
    w1d = _blockdiag4(wz1.astype(bf))
    w2d = _blockdiag4(wz2.astype(bf))
    w3d = _blockdiag4(wz3.astype(bf))
    b1d = jnp.tile(b1, (1, 4))
    b2d = jnp.tile(b2, (1, 4))
    b3d = jnp.tile(b3, (1, 4))
    x4 = x.reshape(R, 1024)

    logits, sums = pl.pallas_call(
        _mlp_body,
        grid=(NB1,),
        in_specs=[
            pl.BlockSpec((RB1, 1024), lambda i: (i, 0)),
            pl.BlockSpec((1024, 256), full),
            pl.BlockSpec((1, 256), full),
            pl.BlockSpec((256, 64), full),
            pl.BlockSpec((1, 64), full),
            pl.BlockSpec((64, 16), full),
            pl.BlockSpec((1, 16), full),
        ],
        out_specs=[
            pl.BlockSpec((16, RB1), lambda i: (0, i)),
            pl.BlockSpec((1, 1, 8), lambda i: (i, 0, 0)),
        ],
        out_shape=[
            jax.ShapeDtypeStruct((16, R), jnp.float32),
            jax.ShapeDtypeStruct((NB1, 1, 8), jnp.float32),
        ],
        compiler_params=pltpu.CompilerParams(
            dimension_semantics=("arbitrary",),
            vmem_limit_bytes=56 * 1024 * 1024,
        ),
    )(x4, w1d, b1d, w2d, b2d, w3d, b3d)

    zd = logits.reshape(D, 128)

    out = pl.pallas_call(
        _norm_body,
        grid=(NB2,),
        in_specs=[
            pl.BlockSpec((RB2, 128), lambda i: (i, 0)),
            pl.BlockSpec((NB1, 1, 8), full3),
        ],
        out_specs=pl.BlockSpec((RB2, 128), lambda i: (i, 0)),
        out_shape=jax.ShapeDtypeStruct((D, 128), jnp.float32),
        compiler_params=pltpu.CompilerParams(
            dimension_semantics=("arbitrary",),
        ),
    )(zd, sums)
    return out.reshape(16, R).T.reshape(B, 4)


# speedup vs baseline: 4.7115x; 4.7115x over previous
import jax
import jax.numpy as jnp
from jax.experimental import pallas as pl
from jax.experimental.pallas import tpu as pltpu

B = 262144
R = B // 4
RB1 = 4096
NB1 = R // RB1
D = B * 4 // 128
RB2 = 4096
NB2 = D // RB2


def _mlp_body(x0_ref, x1_ref, x2_ref, x3_ref, w1_ref, b1_ref, w2_ref,
              b2_ref, w3_ref, b3_ref, logits_ref, sums_ref):
    xb = jnp.concatenate(
        [x0_ref[...], x1_ref[...], x2_ref[...], x3_ref[...]],
        axis=1).astype(jnp.bfloat16)
    h = jnp.dot(xb, w1_ref[...], preferred_element_type=jnp.float32)
    h = jnp.maximum(h + b1_ref[...], 0.0).astype(jnp.bfloat16)
    h = jnp.tanh(jnp.dot(h, w2_ref[...], preferred_element_type=jnp.float32)
                 + b2_ref[...]).astype(jnp.bfloat16)
    z = jnp.dot(h, w3_ref[...], preferred_element_type=jnp.float32) + b3_ref[...]
    zt = jax.lax.transpose(z, (1, 0))
    logits_ref[...] = zt
    sums_ref[...] = jnp.full((1, 1, 8), jnp.sum(jnp.exp(zt)), jnp.float32)


def _norm_body(z_ref, sums_ref, out_ref):
    s = jnp.sum(sums_ref[...]) * 0.125
    out_ref[...] = jnp.exp(z_ref[...]) / s


def _blockdiag4(w):
    k, n = w.shape
    out = jnp.zeros((4 * k, 4 * n), w.dtype)
    for g in range(4):
        out = out.at[g * k:(g + 1) * k, g * n:(g + 1) * n].set(w)
    return out


@jax.jit
def kernel(x, wz1, b1, wz2, b2, wz3, b3):
    full = lambda *_: (0, 0)
    full3 = lambda *_: (0, 0, 0)
    bf = jnp.bfloat16
    w1d = _blockdiag4(wz1.astype(bf))
    w2d = _blockdiag4(wz2.astype(bf))
    w3d = _blockdiag4(wz3.astype(bf))
    b1d = jnp.tile(b1, (1, 4))
    b2d = jnp.tile(b2, (1, 4))
    b3d = jnp.tile(b3, (1, 4))
    logits, sums = pl.pallas_call(
        _mlp_body,
        grid=(NB1,),
        in_specs=[
            pl.BlockSpec((RB1, 256), lambda i: (0 * NB1 + i, 0)),
            pl.BlockSpec((RB1, 256), lambda i: (1 * NB1 + i, 0)),
            pl.BlockSpec((RB1, 256), lambda i: (2 * NB1 + i, 0)),
            pl.BlockSpec((RB1, 256), lambda i: (3 * NB1 + i, 0)),
            pl.BlockSpec((1024, 256), full),
            pl.BlockSpec((1, 256), full),
            pl.BlockSpec((256, 64), full),
            pl.BlockSpec((1, 64), full),
            pl.BlockSpec((64, 16), full),
            pl.BlockSpec((1, 16), full),
        ],
        out_specs=[
            pl.BlockSpec((16, RB1), lambda i: (0, i)),
            pl.BlockSpec((1, 1, 8), lambda i: (i, 0, 0)),
        ],
        out_shape=[
            jax.ShapeDtypeStruct((16, R), jnp.float32),
            jax.ShapeDtypeStruct((NB1, 1, 8), jnp.float32),
        ],
        compiler_params=pltpu.CompilerParams(
            dimension_semantics=("arbitrary",),
            vmem_limit_bytes=56 * 1024 * 1024,
        ),
    )(x, x, x, x, w1d, b1d, w2d, b2d, w3d, b3d)

    zd = logits.reshape(D, 128)

    out = pl.pallas_call(
        _norm_body,
        grid=(NB2,),
        in_specs=[
            pl.BlockSpec((RB2, 128), lambda i: (i, 0)),
            pl.BlockSpec((NB1, 1, 8), full3),
        ],
        out_specs=pl.BlockSpec((RB2, 128), lambda i: (i, 0)),
        out_shape=jax.ShapeDtypeStruct((D, 128), jnp.float32),
        compiler_params=pltpu.CompilerParams(
            dimension_semantics=("arbitrary",),
        ),
    )(zd, sums)
    return out.reshape(4, 4, R).transpose(0, 2, 1).reshape(B, 4)
